# baseline (speedup 1.0000x reference)
import jax
import jax.numpy as jnp
from jax import lax
from jax.experimental import pallas as pl
from jax.experimental.pallas import tpu as pltpu
from jax.experimental.pallas import tpu_sc as plsc

_B = 8


def _sc_gather(idx_hbm, table_hbm, out_hbm, idx_v, row_v, sem):
    wid = lax.axis_index("s") * 2 + lax.axis_index("c")

    @pl.when(wid == 0)
    def _():
        pltpu.sync_copy(idx_hbm, idx_v)
        pltpu.async_copy(table_hbm.at[idx_v], row_v, sem).wait()
        pltpu.sync_copy(row_v, out_hbm)


def _tc_scale(scale_ref, x_ref, o_ref):
    o_ref[...] = x_ref[...] * scale_ref[...][:, None, :]


def kernel(x, stage, embeddings):
    b, c, h, w = x.shape
    xt = jnp.transpose(x, (0, 2, 3, 1)).reshape(b, h * w, c)
    stage_arr = jnp.asarray(stage, jnp.int32).reshape((1,))

    mesh = plsc.VectorSubcoreMesh(core_axis_name="c", subcore_axis_name="s")
    scale = pl.kernel(
        _sc_gather,
        mesh=mesh,
        out_type=jax.ShapeDtypeStruct((1, c), jnp.float32),
        scratch_types=[
            pltpu.VMEM((1,), jnp.int32),
            pltpu.VMEM((1, c), jnp.float32),
            pltpu.SemaphoreType.DMA,
        ],
    )(stage_arr, embeddings)

    out = pl.pallas_call(
        _tc_scale,
        grid=(b // _B,),
        in_specs=[
            pl.BlockSpec((1, c), lambda i: (0, 0)),
            pl.BlockSpec((_B, h * w, c), lambda i: (i, 0, 0)),
        ],
        out_specs=pl.BlockSpec((_B, h * w, c), lambda i: (i, 0, 0)),
        out_shape=jax.ShapeDtypeStruct((b, h * w, c), x.dtype),
        compiler_params=pltpu.CompilerParams(
            dimension_semantics=("arbitrary",),
        ),
    )(scale, xt)
    return out.reshape(b, h, w, c).transpose(0, 3, 1, 2)


# speedup vs baseline: 1.3140x; 1.3140x over previous
import jax
import jax.numpy as jnp
from jax.experimental import pallas as pl
from jax.experimental.pallas import tpu as pltpu

_B = 8


def _scale_kernel(stage_ref, emb_ref, x_ref, o_ref):
    s = stage_ref[0]
    emb = emb_ref[...]
    row = jax.lax.broadcasted_iota(jnp.int32, emb.shape, 0)
    scale = jnp.sum(jnp.where(row == s, emb, 0.0), axis=0)
    o_ref[...] = x_ref[...] * scale[None, None, :]


def kernel(x, stage, embeddings):
    b, c, h, w = x.shape
    xt = jnp.transpose(x, (0, 2, 3, 1)).reshape(b, h * w, c)
    stage_arr = jnp.asarray(stage, jnp.int32).reshape((1,))

    out = pl.pallas_call(
        _scale_kernel,
        grid=(b // _B,),
        in_specs=[
            pl.BlockSpec(memory_space=pltpu.SMEM),
            pl.BlockSpec(embeddings.shape, lambda i: (0, 0)),
            pl.BlockSpec((_B, h * w, c), lambda i: (i, 0, 0)),
        ],
        out_specs=pl.BlockSpec((_B, h * w, c), lambda i: (i, 0, 0)),
        out_shape=jax.ShapeDtypeStruct((b, h * w, c), x.dtype),
        compiler_params=pltpu.CompilerParams(
            dimension_semantics=("arbitrary",),
        ),
    )(stage_arr, embeddings, xt)
    return out.reshape(b, h, w, c).transpose(0, 3, 1, 2)
